# baseline (speedup 1.0000x reference)
import functools

import jax
import jax.numpy as jnp
from jax import lax
from jax.experimental import pallas as pl
from jax.experimental.pallas import tpu as pltpu
from jax.experimental.pallas import tpu_sc as plsc

DIM = 32
ROWS = 16384
COLS = 26
NW = 32
IB = ROWS // NW
SKEW = IB + 1

V = 1000000
J1 = 768
NB1 = V // J1
TAIL = V - NB1 * J1
P1 = J1 + 1

_mesh = plsc.VectorSubcoreMesh(core_axis_name="c", subcore_axis_name="s")


@functools.partial(
    pl.kernel,
    mesh=_mesh,
    out_type=jax.ShapeDtypeStruct((V // 4, 4 * DIM), jnp.float32),
    scratch_types=[
        pltpu.VMEM((DIM, P1), jnp.float32),
        pltpu.VMEM((J1 // 4, 4 * DIM), jnp.float32),
        pltpu.VMEM((TAIL // 4, 4 * DIM), jnp.float32),
    ],
    compiler_params=pltpu.CompilerParams(
        use_tc_tiling_on_sc=True, needs_layout_passes=False
    ),
)
def _relayout(tT_hbm, tail_hbm, tlin_hbm, stage, tbuf, tailbuf):
    wid = lax.axis_index("s") * 2 + lax.axis_index("c")
    i16 = lax.iota(jnp.int32, 16)
    nbk = jnp.where(wid < NB1 - (NB1 // NW) * NW, NB1 // NW + 1, NB1 // NW)

    def transpose(nj):
        def tr(j, carry):
            jv = jnp.full((16,), 0, jnp.int32) + j
            v0 = plsc.load_gather(stage, [i16, jv])
            v1 = plsc.load_gather(stage, [i16 + 16, jv])
            r = j // 4
            c0 = (j % 4) * DIM
            tbuf[r, pl.ds(c0, 16)] = v0
            tbuf[r, pl.ds(c0 + 16, 16)] = v1
            return carry

        lax.fori_loop(0, nj, tr, 0)

    def block(k, carry):
        j0 = pl.multiple_of((k * NW + wid) * J1, 128)
        pltpu.sync_copy(tT_hbm.at[:, pl.ds(j0, J1)], stage.at[:, pl.ds(0, J1)])
        transpose(J1)
        r0 = pl.multiple_of(j0 // 4, 32)
        pltpu.sync_copy(tbuf, tlin_hbm.at[pl.ds(r0, J1 // 4)])
        return carry

    lax.fori_loop(0, nbk, block, 0)

    @pl.when(wid == NW - 1)
    def _tail():
        pltpu.sync_copy(tail_hbm, tailbuf)
        pltpu.sync_copy(
            tailbuf, tlin_hbm.at[pl.ds(NB1 * J1 // 4, TAIL // 4)]
        )


@functools.partial(
    pl.kernel,
    mesh=_mesh,
    out_type=jax.ShapeDtypeStruct((COLS, DIM, ROWS), jnp.float32),
    scratch_types=[
        pltpu.VMEM((IB,), jnp.int32),
        pltpu.VMEM((IB,), jnp.int32),
        pltpu.VMEM((IB, DIM), jnp.float32),
        pltpu.VMEM((IB, DIM), jnp.float32),
        pltpu.VMEM((DIM, SKEW), jnp.float32),
        pltpu.VMEM((DIM, SKEW), jnp.float32),
        pltpu.SemaphoreType.DMA((2,)),
        pltpu.SemaphoreType.DMA((2,)),
    ],
    compiler_params=pltpu.CompilerParams(
        use_tc_tiling_on_sc=False, needs_layout_passes=False
    ),
)
def _emb_lookup(xT_hbm, table_hbm, out_hbm, idx0, idx1, rows0, rows1,
                tb0, tb1, gsem, wsem):
    wid = lax.axis_index("s") * 2 + lax.axis_index("c")
    i0 = wid * IB
    idx_b = (idx0, idx1)
    rows_b = (rows0, rows1)
    tb_b = (tb0, tb1)
    i16 = lax.iota(jnp.int32, 16)

    def fire_gather(q, b):
        idx = idx_b[b]
        pltpu.sync_copy(xT_hbm.at[q, pl.ds(i0, IB)], idx)
        return pltpu.async_copy(table_hbm.at[idx], rows_b[b], gsem.at[b])

    def wait_gather(b):
        pltpu.make_async_copy(
            table_hbm.at[idx_b[b]], rows_b[b], gsem.at[b]
        ).wait()

    def fire_write(q, b):
        return pltpu.async_copy(
            tb_b[b].at[:, pl.ds(0, IB)],
            out_hbm.at[q, :, pl.ds(i0, IB)],
            wsem.at[b],
        )

    def wait_write(q, b):
        pltpu.make_async_copy(
            tb_b[b].at[:, pl.ds(0, IB)],
            out_hbm.at[q, :, pl.ds(i0, IB)],
            wsem.at[b],
        ).wait()

    def transpose(b):
        rows = rows_b[b]
        tb = tb_b[b]

        def pair(j2, carry):
            for u in range(2):
                j = j2 * 2 + u
                jv = jnp.full((16,), 0, jnp.int32) + j
                v0 = rows[j, pl.ds(0, 16)]
                v1 = rows[j, pl.ds(16, 16)]
                plsc.store_scatter(tb, [i16, jv], v0)
                plsc.store_scatter(tb, [i16 + 16, jv], v1)
            return carry

        lax.fori_loop(0, IB // 2, pair, 0)

    fire_gather(0, 0)
    fire_gather(1, 1)

    for u in range(2):
        wait_gather(u)
        transpose(u)
        fire_gather(u + 2, u)
        fire_write(u, u)

    def body(t, carry):
        for u in range(2):
            q = 2 * t + u
            wait_gather(u)
            wait_write(q - 2, u)
            transpose(u)
            fire_gather(q + 2, u)
            fire_write(q, u)
        return carry

    lax.fori_loop(1, 12, body, 0)

    for u in range(2):
        q = 24 + u
        wait_gather(u)
        wait_write(q - 2, u)
        transpose(u)
        fire_write(q, u)
    for u in range(2):
        wait_write(24 + u, u)


def kernel(x, table):
    tail_lin = table[NB1 * J1:].reshape(TAIL // 4, 4 * DIM)
    t_lin = _relayout(table.T, tail_lin)
    outT = _emb_lookup(x.T, t_lin.reshape(V, DIM))
    return outT.transpose(2, 0, 1)


# speedup vs baseline: 2.9478x; 2.9478x over previous
import functools

import jax
import jax.numpy as jnp
from jax import lax
from jax.experimental import pallas as pl
from jax.experimental.pallas import tpu as pltpu
from jax.experimental.pallas import tpu_sc as plsc

DIM = 32
ROWS = 16384
COLS = 26
NW = 32
IB = ROWS // NW
SKEW = IB + 1

V = 1000000
J1 = 768
NB1 = V // J1
TAIL = V - NB1 * J1
P1 = J1

_mesh = plsc.VectorSubcoreMesh(core_axis_name="c", subcore_axis_name="s")


@functools.partial(
    pl.kernel,
    mesh=_mesh,
    out_type=jax.ShapeDtypeStruct((V * DIM,), jnp.float32),
    scratch_types=[
        pltpu.VMEM((DIM * P1,), jnp.float32),
        pltpu.VMEM((DIM * P1,), jnp.float32),
        pltpu.VMEM((J1 * DIM,), jnp.float32),
        pltpu.VMEM((J1 * DIM,), jnp.float32),
        pltpu.VMEM((TAIL * DIM,), jnp.float32),
        pltpu.SemaphoreType.DMA((2,)),
        pltpu.SemaphoreType.DMA((2,)),
    ],
    compiler_params=pltpu.CompilerParams(
        use_tc_tiling_on_sc=True, needs_layout_passes=False
    ),
)
def _relayout(tT_hbm, tail_hbm, tlin_hbm, st0, st1, tb0, tb1, tailbuf,
              isem, osem):
    wid = lax.axis_index("s") * 2 + lax.axis_index("c")
    i16 = lax.iota(jnp.int32, 16)
    base_lo = i16 * P1
    base_hi = base_lo + 16 * P1
    st_b = (st0, st1)
    tb_b = (tb0, tb1)
    nbk = jnp.where(wid < NB1 - (NB1 // NW) * NW, NB1 // NW + 1, NB1 // NW)

    def j0_of(k):
        return pl.multiple_of((k * NW + wid) * J1, 128)

    def fire_in(k, u):
        j0 = j0_of(k)
        for d in range(DIM):
            pltpu.async_copy(
                tT_hbm.at[d, pl.ds(j0, J1)],
                st_b[u].at[pl.ds(d * P1, J1)],
                isem.at[u],
            )

    def wait_in(k, u):
        j0 = j0_of(k)
        for d in range(DIM):
            pltpu.make_async_copy(
                tT_hbm.at[d, pl.ds(j0, J1)],
                st_b[u].at[pl.ds(d * P1, J1)],
                isem.at[u],
            ).wait()

    def fire_out(k, u):
        o0 = pl.multiple_of((k * NW + wid) * (J1 * DIM), 1024)
        pltpu.async_copy(tb_b[u], tlin_hbm.at[pl.ds(o0, J1 * DIM)], osem.at[u])

    def wait_out(k, u):
        o0 = pl.multiple_of((k * NW + wid) * (J1 * DIM), 1024)
        pltpu.make_async_copy(
            tb_b[u], tlin_hbm.at[pl.ds(o0, J1 * DIM)], osem.at[u]
        ).wait()

    def transpose(u):
        st = st_b[u]
        tb = tb_b[u]
        hi16 = i16 + 16

        def tr(j, carry):
            jw0 = i16 + j
            w0 = jnp.where(jw0 >= J1, jw0 - J1, jw0)
            jw1 = jw0 + 16
            w1 = jnp.where(jw1 >= J1, jw1 - J1, jw1)
            v0 = plsc.load_gather(st, [w0 + base_lo])
            v1 = plsc.load_gather(st, [w1 + base_hi])
            plsc.store_scatter(tb, [w0 * DIM + i16], v0)
            plsc.store_scatter(tb, [w1 * DIM + hi16], v1)
            return carry

        lax.fori_loop(0, J1, tr, 0)

    fire_in(0, 0)

    def body(t, carry):
        for u in range(2):
            k = 2 * t + u

            @pl.when(k < nbk)
            def _step():
                wait_in(k, u)

                @pl.when(k + 1 < nbk)
                def _pf():
                    fire_in(k + 1, 1 - u)

                transpose(u)

                @pl.when(k >= 2)
                def _drain():
                    wait_out(k - 2, u)

                fire_out(k, u)

        return carry

    lax.fori_loop(0, 21, body, 0)
    for u in range(2):
        wait_out(0, u)

    @pl.when(wid == NW - 1)
    def _tail():
        pltpu.sync_copy(tail_hbm, tailbuf)
        pltpu.sync_copy(
            tailbuf, tlin_hbm.at[pl.ds(NB1 * J1 * DIM, TAIL * DIM)]
        )


@functools.partial(
    pl.kernel,
    mesh=_mesh,
    out_type=jax.ShapeDtypeStruct((COLS, DIM, ROWS), jnp.float32),
    scratch_types=[
        pltpu.VMEM((IB,), jnp.int32),
        pltpu.VMEM((IB,), jnp.int32),
        pltpu.VMEM((IB, DIM), jnp.float32),
        pltpu.VMEM((IB, DIM), jnp.float32),
        pltpu.VMEM((DIM, SKEW), jnp.float32),
        pltpu.VMEM((DIM, SKEW), jnp.float32),
        pltpu.SemaphoreType.DMA((2,)),
        pltpu.SemaphoreType.DMA((2,)),
    ],
    compiler_params=pltpu.CompilerParams(
        use_tc_tiling_on_sc=False, needs_layout_passes=False
    ),
)
def _emb_lookup(xT_hbm, table_hbm, out_hbm, idx0, idx1, rows0, rows1,
                tb0, tb1, gsem, wsem):
    wid = lax.axis_index("s") * 2 + lax.axis_index("c")
    i0 = wid * IB
    idx_b = (idx0, idx1)
    rows_b = (rows0, rows1)
    tb_b = (tb0, tb1)
    i16 = lax.iota(jnp.int32, 16)

    def fire_gather(q, b):
        idx = idx_b[b]
        pltpu.sync_copy(xT_hbm.at[q, pl.ds(i0, IB)], idx)
        return pltpu.async_copy(table_hbm.at[idx], rows_b[b], gsem.at[b])

    def wait_gather(b):
        pltpu.make_async_copy(
            table_hbm.at[idx_b[b]], rows_b[b], gsem.at[b]
        ).wait()

    def fire_write(q, b):
        return pltpu.async_copy(
            tb_b[b].at[:, pl.ds(0, IB)],
            out_hbm.at[q, :, pl.ds(i0, IB)],
            wsem.at[b],
        )

    def wait_write(q, b):
        pltpu.make_async_copy(
            tb_b[b].at[:, pl.ds(0, IB)],
            out_hbm.at[q, :, pl.ds(i0, IB)],
            wsem.at[b],
        ).wait()

    def transpose(b):
        rows = rows_b[b]
        tb = tb_b[b]

        def pair(j2, carry):
            for u in range(2):
                j = j2 * 2 + u
                jv = jnp.full((16,), 0, jnp.int32) + j
                v0 = rows[j, pl.ds(0, 16)]
                v1 = rows[j, pl.ds(16, 16)]
                plsc.store_scatter(tb, [i16, jv], v0)
                plsc.store_scatter(tb, [i16 + 16, jv], v1)
            return carry

        lax.fori_loop(0, IB // 2, pair, 0)

    fire_gather(0, 0)
    fire_gather(1, 1)

    for u in range(2):
        wait_gather(u)
        transpose(u)
        fire_gather(u + 2, u)
        fire_write(u, u)

    def body(t, carry):
        for u in range(2):
            q = 2 * t + u
            wait_gather(u)
            wait_write(q - 2, u)
            transpose(u)
            fire_gather(q + 2, u)
            fire_write(q, u)
        return carry

    lax.fori_loop(1, 12, body, 0)

    for u in range(2):
        q = 24 + u
        wait_gather(u)
        wait_write(q - 2, u)
        transpose(u)
        fire_write(q, u)
    for u in range(2):
        wait_write(24 + u, u)


def kernel(x, table):
    tail_lin = table[NB1 * J1:].reshape(TAIL * DIM)
    t_lin = _relayout(table.T, tail_lin)
    outT = _emb_lookup(x.T, t_lin.reshape(V, DIM))
    return outT.transpose(2, 0, 1)


# speedup vs baseline: 3.4457x; 1.1689x over previous
import functools

import jax
import jax.numpy as jnp
from jax import lax
from jax.experimental import pallas as pl
from jax.experimental.pallas import tpu as pltpu
from jax.experimental.pallas import tpu_sc as plsc

DIM = 32
ROWS = 16384
COLS = 26
NW = 32
IB = ROWS // NW
SKEW = IB + 1

V = 1000000
J1 = 768
NB1 = V // J1
TAIL = V - NB1 * J1
P1 = J1

_mesh = plsc.VectorSubcoreMesh(core_axis_name="c", subcore_axis_name="s")


@functools.partial(
    pl.kernel,
    mesh=_mesh,
    out_type=jax.ShapeDtypeStruct((V * DIM,), jnp.float32),
    scratch_types=[
        pltpu.VMEM((DIM * P1,), jnp.float32),
        pltpu.VMEM((DIM * P1,), jnp.float32),
        pltpu.VMEM((J1 * DIM,), jnp.float32),
        pltpu.VMEM((J1 * DIM,), jnp.float32),
        pltpu.VMEM((TAIL * DIM,), jnp.float32),
        pltpu.SemaphoreType.DMA((2,)),
        pltpu.SemaphoreType.DMA((2,)),
    ],
    compiler_params=pltpu.CompilerParams(
        use_tc_tiling_on_sc=True, needs_layout_passes=False
    ),
)
def _relayout(tT_hbm, tail_hbm, tlin_hbm, st0, st1, tb0, tb1, tailbuf,
              isem, osem):
    wid = lax.axis_index("s") * 2 + lax.axis_index("c")
    i16 = lax.iota(jnp.int32, 16)
    base_lo = i16 * P1
    base_hi = base_lo + 16 * P1
    st_b = (st0, st1)
    tb_b = (tb0, tb1)
    nbk = jnp.where(wid < NB1 - (NB1 // NW) * NW, NB1 // NW + 1, NB1 // NW)

    def j0_of(k):
        return pl.multiple_of((k * NW + wid) * J1, 128)

    def fire_in(k, u):
        j0 = j0_of(k)
        for d in range(DIM):
            pltpu.async_copy(
                tT_hbm.at[d, pl.ds(j0, J1)],
                st_b[u].at[pl.ds(d * P1, J1)],
                isem.at[u],
            )

    def wait_in(k, u):
        j0 = j0_of(k)
        for d in range(DIM):
            pltpu.make_async_copy(
                tT_hbm.at[d, pl.ds(j0, J1)],
                st_b[u].at[pl.ds(d * P1, J1)],
                isem.at[u],
            ).wait()

    def fire_out(k, u):
        o0 = pl.multiple_of((k * NW + wid) * (J1 * DIM), 1024)
        pltpu.async_copy(tb_b[u], tlin_hbm.at[pl.ds(o0, J1 * DIM)], osem.at[u])

    def wait_out(k, u):
        o0 = pl.multiple_of((k * NW + wid) * (J1 * DIM), 1024)
        pltpu.make_async_copy(
            tb_b[u], tlin_hbm.at[pl.ds(o0, J1 * DIM)], osem.at[u]
        ).wait()

    def transpose(u):
        st = st_b[u]
        tb = tb_b[u]
        hi16 = i16 + 16

        def tr(j, carry):
            jw0 = i16 + j
            w0 = jnp.where(jw0 >= J1, jw0 - J1, jw0)
            jw1 = jw0 + 16
            w1 = jnp.where(jw1 >= J1, jw1 - J1, jw1)
            v0 = plsc.load_gather(st, [w0 + base_lo])
            v1 = plsc.load_gather(st, [w1 + base_hi])
            plsc.store_scatter(tb, [w0 * DIM + i16], v0)
            plsc.store_scatter(tb, [w1 * DIM + hi16], v1)
            return carry

        lax.fori_loop(0, J1, tr, 0)

    fire_in(0, 0)

    def body(t, carry):
        for u in range(2):
            k = 2 * t + u

            @pl.when(k < nbk)
            def _step():
                wait_in(k, u)

                @pl.when(k + 1 < nbk)
                def _pf():
                    fire_in(k + 1, 1 - u)

                transpose(u)

                @pl.when(k >= 2)
                def _drain():
                    wait_out(k - 2, u)

                fire_out(k, u)

        return carry

    lax.fori_loop(0, 21, body, 0)
    for u in range(2):
        wait_out(0, u)

    @pl.when(wid == NW - 1)
    def _tail():
        pltpu.sync_copy(tail_hbm, tailbuf)
        pltpu.sync_copy(
            tailbuf, tlin_hbm.at[pl.ds(NB1 * J1 * DIM, TAIL * DIM)]
        )


@functools.partial(
    pl.kernel,
    mesh=_mesh,
    out_type=jax.ShapeDtypeStruct((COLS, DIM // 8, ROWS // 128, 8, 128),
                                  jnp.float32),
    scratch_types=[
        pltpu.VMEM((IB,), jnp.int32),
        pltpu.VMEM((IB,), jnp.int32),
        pltpu.VMEM((IB, DIM), jnp.float32),
        pltpu.VMEM((IB, DIM), jnp.float32),
        pltpu.VMEM((DIM, SKEW), jnp.float32),
        pltpu.VMEM((DIM, SKEW), jnp.float32),
        pltpu.SemaphoreType.DMA((2,)),
        pltpu.SemaphoreType.DMA((2,)),
    ],
    compiler_params=pltpu.CompilerParams(
        use_tc_tiling_on_sc=False, needs_layout_passes=False
    ),
)
def _emb_lookup(xT_hbm, table_hbm, out_hbm, idx0, idx1, rows0, rows1,
                tb0, tb1, gsem, wsem):
    wid = lax.axis_index("s") * 2 + lax.axis_index("c")
    i0 = wid * IB
    idx_b = (idx0, idx1)
    rows_b = (rows0, rows1)
    tb_b = (tb0, tb1)
    i16 = lax.iota(jnp.int32, 16)

    def fire_gather(q, b):
        idx = idx_b[b]
        pltpu.sync_copy(xT_hbm.at[q, pl.ds(i0, IB)], idx)
        return pltpu.async_copy(table_hbm.at[idx], rows_b[b], gsem.at[b])

    def wait_gather(b):
        pltpu.make_async_copy(
            table_hbm.at[idx_b[b]], rows_b[b], gsem.at[b]
        ).wait()

    tblk = wid * (IB // 128)

    def fire_write(q, b):
        for g in range(DIM // 8):
            for t in range(IB // 128):
                pltpu.async_copy(
                    tb_b[b].at[pl.ds(8 * g, 8), pl.ds(128 * t, 128)],
                    out_hbm.at[q, g, tblk + t],
                    wsem.at[b],
                )

    def wait_write(q, b):
        for g in range(DIM // 8):
            for t in range(IB // 128):
                pltpu.make_async_copy(
                    tb_b[b].at[pl.ds(8 * g, 8), pl.ds(128 * t, 128)],
                    out_hbm.at[q, g, tblk + t],
                    wsem.at[b],
                ).wait()

    def transpose(b):
        rows = rows_b[b]
        tb = tb_b[b]

        def pair(j2, carry):
            for u in range(2):
                j = j2 * 2 + u
                jv = jnp.full((16,), 0, jnp.int32) + j
                v0 = rows[j, pl.ds(0, 16)]
                v1 = rows[j, pl.ds(16, 16)]
                plsc.store_scatter(tb, [i16, jv], v0)
                plsc.store_scatter(tb, [i16 + 16, jv], v1)
            return carry

        lax.fori_loop(0, IB // 2, pair, 0)

    fire_gather(0, 0)
    fire_gather(1, 1)

    for u in range(2):
        wait_gather(u)
        transpose(u)
        fire_gather(u + 2, u)
        fire_write(u, u)

    def body(t, carry):
        for u in range(2):
            q = 2 * t + u
            wait_gather(u)
            wait_write(q - 2, u)
            transpose(u)
            fire_gather(q + 2, u)
            fire_write(q, u)
        return carry

    lax.fori_loop(1, 12, body, 0)

    for u in range(2):
        q = 24 + u
        wait_gather(u)
        wait_write(q - 2, u)
        transpose(u)
        fire_write(q, u)
    for u in range(2):
        wait_write(24 + u, u)


def kernel(x, table):
    tail_lin = table[NB1 * J1:].reshape(TAIL * DIM)
    t_lin = _relayout(table.T, tail_lin)
    out5 = _emb_lookup(x.T, t_lin.reshape(V, DIM))
    return out5.transpose(2, 4, 0, 1, 3).reshape(ROWS, COLS, DIM)
